# baseline (speedup 1.0000x reference)
import functools

import jax
import jax.numpy as jnp
from jax import lax
from jax.experimental import pallas as pl
from jax.experimental.pallas import tpu as pltpu
from jax.experimental.pallas import tpu_sc as plsc

_EMBED = 64
_LANES = 16
_NCHUNK = 2


def _pack_kernel(a_ref, o_ref):
    h = pl.program_id(0) % 2

    @pl.when(h == 0)
    def _():
        o_ref[:, 0:_EMBED] = a_ref[...]

    @pl.when(h == 1)
    def _():
        o_ref[:, _EMBED:2 * _EMBED] = a_ref[...]


@functools.lru_cache(maxsize=None)
def _pack_table(V):
    blk = 2000
    assert V % (2 * blk) == 0
    n = V // (2 * blk)
    return pl.pallas_call(
        _pack_kernel,
        grid=(2 * n,),
        in_specs=[pl.BlockSpec((blk, _EMBED),
                               lambda i, n=n: ((i % 2) * n + i // 2, 0))],
        out_specs=pl.BlockSpec((blk, 2 * _EMBED), lambda i: (i // 2, 0)),
        out_shape=jax.ShapeDtypeStruct((V // 2, 2 * _EMBED), jnp.float32),
    )


@functools.lru_cache(maxsize=None)
def _build(B):
    info = plsc.get_sparse_core_info()
    nc, ns = info.num_cores, info.num_subcores
    nw = nc * ns
    assert B % (nw * _NCHUNK * _LANES) == 0
    P = B // nw
    H = P // _NCHUNK

    mesh = plsc.VectorSubcoreMesh(core_axis_name="c", subcore_axis_name="s")

    @functools.partial(
        pl.kernel,
        mesh=mesh,
        out_type=jax.ShapeDtypeStruct((B,), jnp.float32),
        compiler_params=pltpu.CompilerParams(use_tc_tiling_on_sc=True),
        scratch_types=[
            pltpu.VMEM((H,), jnp.int32),
            pltpu.VMEM((H,), jnp.int32),
            pltpu.VMEM((H,), jnp.int32),
            pltpu.VMEM((H,), jnp.int32),
            pltpu.VMEM((H,), jnp.int32),
            pltpu.VMEM((H,), jnp.int32),
            pltpu.VMEM((H, 2 * _EMBED), jnp.float32),
            pltpu.VMEM((H, 2 * _EMBED), jnp.float32),
            pltpu.VMEM((H,), jnp.float32),
            pltpu.VMEM((H,), jnp.float32),
            pltpu.VMEM((H,), jnp.float32),
            pltpu.SemaphoreType.DMA,
            pltpu.SemaphoreType.DMA,
            pltpu.SemaphoreType.DMA,
            pltpu.SemaphoreType.DMA,
        ],
    )
    def k(ti_hbm, ni_hbm, tih_hbm, nih_hbm, toff_hbm, noff_hbm,
          te_hbm, ne_hbm, tb_hbm, nb_hbm, out_hbm,
          ti_v, ni_v, tih_v, nih_v, toff_v, noff_v,
          trow_v, nrow_v, tb_v, nb_v, out_v,
          sem_t, sem_n, sem_tb, sem_nb):
        wid = lax.axis_index("s") * nc + lax.axis_index("c")
        base = wid * P

        lanes = lax.iota(jnp.int32, _LANES)
        dnums = lax.GatherDimensionNumbers(
            offset_dims=(), collapsed_slice_dims=(0,), start_index_map=(0,))

        def shuffle(v, idx):
            return lax.gather(v, idx[:, None], dnums, slice_sizes=(1,),
                              mode=lax.GatherScatterMode.PROMISE_IN_BOUNDS)

        def hsum(v):
            for k in (8, 4, 2, 1):
                v = v + shuffle(v, lanes ^ k)
            return v

        def chunk(h, carry):
            cbase = base + h * H
            pltpu.sync_copy(ti_hbm.at[pl.ds(cbase, H)], ti_v)
            pltpu.sync_copy(ni_hbm.at[pl.ds(cbase, H)], ni_v)
            pltpu.sync_copy(tih_hbm.at[pl.ds(cbase, H)], tih_v)
            pltpu.sync_copy(nih_hbm.at[pl.ds(cbase, H)], nih_v)
            pltpu.sync_copy(toff_hbm.at[pl.ds(cbase, H)], toff_v)
            pltpu.sync_copy(noff_hbm.at[pl.ds(cbase, H)], noff_v)
            ct = pltpu.async_copy(te_hbm.at[tih_v], trow_v, sem_t)
            cn = pltpu.async_copy(ne_hbm.at[nih_v], nrow_v, sem_n)
            ctb = pltpu.async_copy(tb_hbm.at[ti_v], tb_v, sem_tb)
            cnb = pltpu.async_copy(nb_hbm.at[ni_v], nb_v, sem_nb)
            ct.wait()
            cn.wait()
            ctb.wait()
            cnb.wait()

            def group(g, c):
                r0 = g * _LANES
                tof = toff_v[pl.ds(r0, _LANES)]
                nof = noff_v[pl.ds(r0, _LANES)]
                res = jnp.zeros((_LANES,), jnp.float32)
                for j in range(_LANES):
                    r = r0 + j
                    to = tof[j]
                    no = nof[j]
                    acc = (trow_v[r, pl.ds(to, _LANES)]
                           * nrow_v[r, pl.ds(no, _LANES)])
                    for q in range(1, _EMBED // _LANES):
                        acc = acc + (trow_v[r, pl.ds(to + q * _LANES, _LANES)]
                                     * nrow_v[r, pl.ds(no + q * _LANES, _LANES)])
                    res = jnp.where(lanes == j, hsum(acc), res)
                res = res - (tb_v[pl.ds(r0, _LANES)] + nb_v[pl.ds(r0, _LANES)])
                out_v[pl.ds(r0, _LANES)] = 1.0 / (1.0 + jnp.exp(-res))
                return c

            lax.fori_loop(0, H // _LANES, group, 0)
            pltpu.sync_copy(out_v, out_hbm.at[pl.ds(cbase, H)])
            return carry

        lax.fori_loop(0, _NCHUNK, chunk, 0)

    return k


def kernel(x, track_embedding, name_embedding, track_bias, name_bias):
    ti = x[:, 0].astype(jnp.int32)
    ni = x[:, 1].astype(jnp.int32)
    half_t = track_embedding.shape[0] // 2
    half_n = name_embedding.shape[0] // 2
    tih = jnp.where(ti < half_t, ti, ti - half_t)
    nih = jnp.where(ni < half_n, ni, ni - half_n)
    toff = jnp.where(ti < half_t, 0, _EMBED)
    noff = jnp.where(ni < half_n, 0, _EMBED)
    te = _pack_table(track_embedding.shape[0])(track_embedding)
    ne = _pack_table(name_embedding.shape[0])(name_embedding)
    tb = -track_bias[:, 0]
    nb = -name_bias[:, 0]
    return _build(x.shape[0])(ti, ni, tih, nih, toff, noff, te, ne, tb, nb)


# speedup vs baseline: 1.5729x; 1.5729x over previous
import functools

import jax
import jax.numpy as jnp
from jax import lax
from jax.experimental import pallas as pl
from jax.experimental.pallas import tpu as pltpu
from jax.experimental.pallas import tpu_sc as plsc

_EMBED = 64
_LANES = 16


@functools.lru_cache(maxsize=None)
def _build(B):
    info = plsc.get_sparse_core_info()
    nc, ns = info.num_cores, info.num_subcores
    nw = nc * ns
    assert B % nw == 0
    P = B // nw

    mesh = plsc.VectorSubcoreMesh(core_axis_name="c", subcore_axis_name="s")

    @functools.partial(
        pl.kernel,
        mesh=mesh,
        out_type=jax.ShapeDtypeStruct((B,), jnp.float32),
        compiler_params=pltpu.CompilerParams(use_tc_tiling_on_sc=False),
        scratch_types=[
            pltpu.VMEM((P,), jnp.int32),
            pltpu.VMEM((P,), jnp.int32),
            pltpu.VMEM((P, _EMBED), jnp.float32),
            pltpu.VMEM((P, _EMBED), jnp.float32),
            pltpu.VMEM((P,), jnp.float32),
            pltpu.VMEM((P,), jnp.float32),
            pltpu.VMEM((P,), jnp.float32),
            pltpu.SemaphoreType.DMA,
            pltpu.SemaphoreType.DMA,
            pltpu.SemaphoreType.DMA,
            pltpu.SemaphoreType.DMA,
        ],
    )
    def k(ti_hbm, ni_hbm, te_hbm, ne_hbm, tb_hbm, nb_hbm, out_hbm,
          ti_v, ni_v, trow_v, nrow_v, tb_v, nb_v, out_v,
          sem_t, sem_n, sem_tb, sem_nb):
        wid = lax.axis_index("s") * nc + lax.axis_index("c")
        base = wid * P
        pltpu.sync_copy(ti_hbm.at[pl.ds(base, P)], ti_v)
        pltpu.sync_copy(ni_hbm.at[pl.ds(base, P)], ni_v)
        ct = pltpu.async_copy(te_hbm.at[ti_v], trow_v, sem_t)
        cn = pltpu.async_copy(ne_hbm.at[ni_v], nrow_v, sem_n)
        ctb = pltpu.async_copy(tb_hbm.at[ti_v], tb_v, sem_tb)
        cnb = pltpu.async_copy(nb_hbm.at[ni_v], nb_v, sem_nb)
        ct.wait()
        cn.wait()
        ctb.wait()
        cnb.wait()

        lanes = lax.iota(jnp.int32, _LANES)
        dnums = lax.GatherDimensionNumbers(
            offset_dims=(), collapsed_slice_dims=(0,), start_index_map=(0,))

        def shuffle(v, idx):
            return lax.gather(v, idx[:, None], dnums, slice_sizes=(1,),
                              mode=lax.GatherScatterMode.PROMISE_IN_BOUNDS)

        def hsum(v):
            for k in (8, 4, 2, 1):
                v = v + shuffle(v, lanes ^ k)
            return v

        def group(g, carry):
            r0 = g * _LANES
            res = jnp.zeros((_LANES,), jnp.float32)
            for j in range(_LANES):
                r = r0 + j
                acc = trow_v[r, pl.ds(0, _LANES)] * nrow_v[r, pl.ds(0, _LANES)]
                for q in range(1, _EMBED // _LANES):
                    acc = acc + (trow_v[r, pl.ds(q * _LANES, _LANES)]
                                 * nrow_v[r, pl.ds(q * _LANES, _LANES)])
                res = jnp.where(lanes == j, hsum(acc), res)
            res = res - (tb_v[pl.ds(r0, _LANES)] + nb_v[pl.ds(r0, _LANES)])
            out_v[pl.ds(r0, _LANES)] = 1.0 / (1.0 + jnp.exp(-res))
            return carry

        lax.fori_loop(0, P // _LANES, group, 0)
        pltpu.sync_copy(out_v, out_hbm.at[pl.ds(base, P)])

    return k


def kernel(x, track_embedding, name_embedding, track_bias, name_bias):
    ti = x[:, 0].astype(jnp.int32)
    ni = x[:, 1].astype(jnp.int32)
    tb = -track_bias[:, 0]
    nb = -name_bias[:, 0]
    return _build(x.shape[0])(ti, ni, track_embedding, name_embedding, tb, nb)
